# baseline (speedup 1.0000x reference)
import functools

import jax
import jax.numpy as jnp
from jax import lax
from jax.experimental import pallas as pl
from jax.experimental.pallas import tpu as pltpu

TOKENS = 8192
D_MODEL = 768
FACTORS = 32768
TOPK = 64

TB = 512
FB = 2048


def _matmul_body(x_ref, w_ref, b_ref, out_ref):
    xc = x_ref[...] - b_ref[...]
    out_ref[...] = jnp.dot(xc, w_ref[...], preferred_element_type=jnp.float32)


def _pre_acts(x, W_enc, b_pre):
    return pl.pallas_call(
        _matmul_body,
        grid=(FACTORS // FB, TOKENS // TB),
        in_specs=[
            pl.BlockSpec((TB, D_MODEL), lambda f, t: (t, 0)),
            pl.BlockSpec((D_MODEL, FB), lambda f, t: (0, f)),
            pl.BlockSpec((1, D_MODEL), lambda f, t: (0, 0)),
        ],
        out_specs=pl.BlockSpec((TB, FB), lambda f, t: (t, f)),
        out_shape=jax.ShapeDtypeStruct((TOKENS, FACTORS), jnp.float32),
        compiler_params=pltpu.CompilerParams(
            dimension_semantics=("arbitrary", "arbitrary"),
        ),
    )(x, W_enc, b_pre.reshape(1, D_MODEL))


def kernel(x, W_enc, W_dec, b_pre):
    pre = _pre_acts(x, W_enc, b_pre)
    vals, idx = lax.top_k(pre, TOPK)
    row_idx = jnp.arange(TOKENS)[:, None]
    acts = jnp.zeros((TOKENS, FACTORS), jnp.float32).at[row_idx, idx].set(vals)
    x_hat = acts @ W_dec + b_pre
    return (x_hat, acts)


# speedup vs baseline: 5.2803x; 5.2803x over previous
import jax
import numpy as np
import jax.numpy as jnp
from jax import lax
from jax.experimental import pallas as pl
from jax.experimental.pallas import tpu as pltpu
from jax.experimental.pallas import tpu_sc as plsc

TOKENS = 8192
D_MODEL = 768
FACTORS = 32768
TOPK = 64

TB = 512
FB = 2048


def _matmul_body(x_ref, w_ref, b_ref, out_ref):
    xc = x_ref[...] - b_ref[...]
    out_ref[...] = jnp.dot(xc, w_ref[...], preferred_element_type=jnp.float32)


def _pre_acts(x, W_enc, b_pre):
    return pl.pallas_call(
        _matmul_body,
        grid=(FACTORS // FB, TOKENS // TB),
        in_specs=[
            pl.BlockSpec((TB, D_MODEL), lambda f, t: (t, 0)),
            pl.BlockSpec((D_MODEL, FB), lambda f, t: (0, f)),
            pl.BlockSpec((1, D_MODEL), lambda f, t: (0, 0)),
        ],
        out_specs=pl.BlockSpec((TB, FB), lambda f, t: (t, f)),
        out_shape=jax.ShapeDtypeStruct((TOKENS, FACTORS), jnp.float32),
        compiler_params=pltpu.CompilerParams(
            dimension_semantics=("arbitrary", "arbitrary"),
        ),
    )(x, W_enc, b_pre.reshape(1, D_MODEL))


KB = 4096


def _dec_body(a_ref, w_ref, b_ref, out_ref):
    k = pl.program_id(1)

    @pl.when(k == 0)
    def _():
        out_ref[...] = jnp.broadcast_to(b_ref[...], out_ref.shape)

    out_ref[...] += jnp.dot(
        a_ref[...], w_ref[...], preferred_element_type=jnp.float32
    )


def _decode(acts, W_dec, b_pre):
    return pl.pallas_call(
        _dec_body,
        grid=(TOKENS // TB, FACTORS // KB),
        in_specs=[
            pl.BlockSpec((TB, KB), lambda t, k: (t, k)),
            pl.BlockSpec((KB, D_MODEL), lambda t, k: (k, 0)),
            pl.BlockSpec((1, D_MODEL), lambda t, k: (0, 0)),
        ],
        out_specs=pl.BlockSpec((TB, D_MODEL), lambda t, k: (t, 0)),
        out_shape=jax.ShapeDtypeStruct((TOKENS, D_MODEL), jnp.float32),
        compiler_params=pltpu.CompilerParams(
            dimension_semantics=("arbitrary", "arbitrary"),
        ),
    )(acts, W_dec, b_pre.reshape(1, D_MODEL))


SC_CORES = 2
SC_SUBCORES = 16
LANES = 16
NW = SC_CORES * SC_SUBCORES
ROWS_PER_W = TOKENS // NW
NVEC = FACTORS // LANES
GROUPS = 256
GVREGS = GROUPS // LANES
VPG = NVEC // GROUPS * LANES
CAP = 512

_SIGN = np.uint32(0x80000000)
_MANT = np.uint32(0x7FFFFFFF)


def _mkey(v):
    b = lax.bitcast_convert_type(v, jnp.uint32)
    neg = (b >> 31) == 1
    return jnp.where(neg, ~b, b | _SIGN)


def _inv_key(u):
    is_pos = (u >> 31) == 1
    b = jnp.where(is_pos, u & _MANT, ~u)
    return lax.bitcast_convert_type(b, jnp.float32)


def _sc_body(pre, acts, row_v, gk_v, cidx_v, ckey_v, selidx_v, act_row):
    wid = lax.axis_index("s") * SC_CORES + lax.axis_index("c")
    r0 = wid * ROWS_PER_W
    lanes = lax.iota(jnp.int32, LANES)
    zero16f = jnp.zeros((LANES,), jnp.float32)
    zero16i = jnp.zeros((LANES,), jnp.int32)

    def _z_act(i, c):
        act_row[pl.ds(i * LANES, LANES)] = zero16f
        return c

    lax.fori_loop(0, NVEC, _z_act, 0)

    def _z_cidx(i, c):
        cidx_v[pl.ds(i * LANES, LANES)] = zero16i
        return c

    lax.fori_loop(0, CAP // LANES, _z_cidx, 0)

    def _z_sel(i, c):
        selidx_v[pl.ds(i * LANES, LANES)] = zero16i
        return c

    lax.fori_loop(0, TOPK // LANES, _z_sel, 0)

    def row_body(i, carry):
        r = r0 + i
        pltpu.sync_copy(pre.at[r], row_v)

        def gblock(b, c):
            def gmax(j, acc):
                return jnp.maximum(
                    acc, row_v[pl.ds((b * 128 + j) * LANES, LANES)]
                )

            acc = lax.fori_loop(
                1, 128, gmax, row_v[pl.ds(b * 128 * LANES, LANES)]
            )
            gk_v[pl.ds(b * LANES, LANES)] = _mkey(acc)
            return c

        lax.fori_loop(0, GVREGS, gblock, 0)

        def bs1(_, lohi):
            lo, hi = lohi
            mid = lo + ((hi - lo) >> 1) + np.uint32(1)

            def cnt(j, acc):
                m = gk_v[pl.ds(j * LANES, LANES)] >= mid
                return acc + jnp.where(m, 1, 0)

            c = jnp.sum(lax.fori_loop(0, GVREGS, cnt, zero16i))
            ok = c >= TOPK
            return (jnp.where(ok, mid, lo), jnp.where(ok, hi, mid - 1))

        tau_m, _ = lax.fori_loop(
            0, 32, bs1, (np.uint32(0), np.uint32(0xFFFFFFFF))
        )
        tau_f = _inv_key(tau_m)

        def ext(i8, pos):
            for k in range(8):
                i2 = i8 * 8 + k
                v = row_v[pl.ds(i2 * LANES, LANES)]
                m = v >= tau_f
                iv = lanes + i2 * LANES
                plsc.store_compressed(cidx_v.at[pl.ds(pos, LANES)], iv, mask=m)
                c = jnp.max(plsc.all_reduce_population_count(m))
                pos = jnp.minimum(pos + c, CAP - LANES)
            return pos

        pos = lax.fori_loop(0, NVEC // 8, ext, 0)
        vn = (pos + LANES - 1) // LANES

        def mk(j, c):
            iv = cidx_v[pl.ds(j * LANES, LANES)]
            vals = plsc.load_gather(row_v, [iv])
            valid = (lanes + j * LANES) < pos
            ckey_v[pl.ds(j * LANES, LANES)] = jnp.where(
                valid, _mkey(vals), np.uint32(0)
            )
            return c

        lax.fori_loop(0, vn, mk, 0)

        def bs2(_, lohi):
            lo, hi = lohi
            mid = lo + ((hi - lo) >> 1) + np.uint32(1)

            def cnt(j, acc):
                m = ckey_v[pl.ds(j * LANES, LANES)] >= mid
                return acc + jnp.where(m, 1, 0)

            c = jnp.sum(lax.fori_loop(0, vn, cnt, zero16i))
            ok = c >= TOPK
            return (jnp.where(ok, mid, lo), jnp.where(ok, hi, mid - 1))

        tau_s, _ = lax.fori_loop(
            0, 32, bs2, (np.uint32(0), np.uint32(0xFFFFFFFF))
        )

        def cgt(j, acc):
            m = ckey_v[pl.ds(j * LANES, LANES)] > tau_s
            return acc + jnp.where(m, 1, 0)

        gt_cnt = jnp.sum(lax.fori_loop(0, vn, cgt, zero16i))
        need = TOPK - gt_cnt

        def selp(j, carry2):
            outpos, eqc = carry2
            kk = ckey_v[pl.ds(j * LANES, LANES)]
            gt = kk > tau_s
            eq = kk == tau_s
            ecs = plsc.cumsum(jnp.where(eq, 1, 0))
            take = eq & ((ecs + eqc) <= need)
            sel = gt | take
            iv = cidx_v[pl.ds(j * LANES, LANES)]
            plsc.store_compressed(selidx_v.at[pl.ds(outpos, LANES)], iv, mask=sel)
            outpos = outpos + jnp.max(plsc.all_reduce_population_count(sel))
            eqc = eqc + jnp.max(ecs)
            return (outpos, eqc)

        lax.fori_loop(0, vn, selp, (0, 0))

        def wsel(j, c):
            iv = selidx_v[pl.ds(j * LANES, LANES)]
            vv = plsc.load_gather(row_v, [iv])
            plsc.store_scatter(act_row, [iv], vv)
            return c

        lax.fori_loop(0, TOPK // LANES, wsel, 0)
        pltpu.sync_copy(act_row, acts.at[r])

        def zsel(j, c):
            iv = selidx_v[pl.ds(j * LANES, LANES)]
            plsc.store_scatter(act_row, [iv], zero16f)
            return c

        lax.fori_loop(0, TOPK // LANES, zsel, 0)
        return carry

    lax.fori_loop(0, ROWS_PER_W, row_body, 0)


def _sc_select(pre):
    mesh = plsc.VectorSubcoreMesh(
        core_axis_name="c", subcore_axis_name="s"
    )
    return pl.kernel(
        _sc_body,
        out_type=jax.ShapeDtypeStruct((TOKENS, FACTORS), jnp.float32),
        mesh=mesh,
        compiler_params=pltpu.CompilerParams(needs_layout_passes=False),
        scratch_types=[
            pltpu.VMEM((FACTORS,), jnp.float32),
            pltpu.VMEM((GROUPS,), jnp.uint32),
            pltpu.VMEM((CAP,), jnp.int32),
            pltpu.VMEM((CAP,), jnp.uint32),
            pltpu.VMEM((TOPK,), jnp.int32),
            pltpu.VMEM((FACTORS,), jnp.float32),
        ],
    )(pre)


def kernel(x, W_enc, W_dec, b_pre):
    pre = _pre_acts(x, W_enc, b_pre)
    acts = _sc_select(pre)
    x_hat = _decode(acts, W_dec, b_pre)
    return (x_hat, acts)


# speedup vs baseline: 10.9818x; 2.0798x over previous
import jax
import numpy as np
import jax.numpy as jnp
from jax import lax
from jax.experimental import pallas as pl
from jax.experimental.pallas import tpu as pltpu
from jax.experimental.pallas import tpu_sc as plsc

TOKENS = 8192
D_MODEL = 768
FACTORS = 32768
TOPK = 64

TB = 512
FB = 2048


def _matmul_body(x_ref, w_ref, b_ref, out_ref):
    xc = x_ref[...] - b_ref[...]
    out_ref[...] = jnp.dot(xc, w_ref[...], preferred_element_type=jnp.float32)


def _pre_acts(x, W_enc, b_pre):
    return pl.pallas_call(
        _matmul_body,
        grid=(FACTORS // FB, TOKENS // TB),
        in_specs=[
            pl.BlockSpec((TB, D_MODEL), lambda f, t: (t, 0)),
            pl.BlockSpec((D_MODEL, FB), lambda f, t: (0, f)),
            pl.BlockSpec((1, D_MODEL), lambda f, t: (0, 0)),
        ],
        out_specs=pl.BlockSpec((TB, FB), lambda f, t: (t, f)),
        out_shape=jax.ShapeDtypeStruct((TOKENS, FACTORS), jnp.float32),
        compiler_params=pltpu.CompilerParams(
            dimension_semantics=("arbitrary", "arbitrary"),
        ),
    )(x, W_enc, b_pre.reshape(1, D_MODEL))


KB = 4096


def _dec_body(a_ref, w_ref, b_ref, out_ref):
    k = pl.program_id(1)

    @pl.when(k == 0)
    def _():
        out_ref[...] = jnp.broadcast_to(b_ref[...], out_ref.shape)

    out_ref[...] += jnp.dot(
        a_ref[...], w_ref[...], preferred_element_type=jnp.float32
    )


def _decode(acts, W_dec, b_pre):
    return pl.pallas_call(
        _dec_body,
        grid=(TOKENS // TB, FACTORS // KB),
        in_specs=[
            pl.BlockSpec((TB, KB), lambda t, k: (t, k)),
            pl.BlockSpec((KB, D_MODEL), lambda t, k: (k, 0)),
            pl.BlockSpec((1, D_MODEL), lambda t, k: (0, 0)),
        ],
        out_specs=pl.BlockSpec((TB, D_MODEL), lambda t, k: (t, 0)),
        out_shape=jax.ShapeDtypeStruct((TOKENS, D_MODEL), jnp.float32),
        compiler_params=pltpu.CompilerParams(
            dimension_semantics=("arbitrary", "arbitrary"),
        ),
    )(acts, W_dec, b_pre.reshape(1, D_MODEL))


SC_CORES = 2
SC_SUBCORES = 16
LANES = 16
NW = SC_CORES * SC_SUBCORES
ROWS_PER_W = TOKENS // NW
NVEC = FACTORS // LANES
GROUPS = 256
GVREGS = GROUPS // LANES
VPG = NVEC // GROUPS * LANES
CAP = 512

_SIGN = np.uint32(0x80000000)
_MANT = np.uint32(0x7FFFFFFF)


def _mkey(v):
    b = lax.bitcast_convert_type(v, jnp.uint32)
    neg = (b >> 31) == 1
    return jnp.where(neg, ~b, b | _SIGN)


def _inv_key(u):
    is_pos = (u >> 31) == 1
    b = jnp.where(is_pos, u & _MANT, ~u)
    return lax.bitcast_convert_type(b, jnp.float32)


def _sc_body(pre, acts, row_a, row_b, bmax_v, gk_v, cidx_v, ckey_v,
             selidx_v, selidx_p, act_row, sem_a, sem_b, sem_o):
    wid = lax.axis_index("s") * SC_CORES + lax.axis_index("c")
    r0 = wid * ROWS_PER_W
    lanes = lax.iota(jnp.int32, LANES)
    zero16f = jnp.zeros((LANES,), jnp.float32)
    zero16i = jnp.zeros((LANES,), jnp.int32)

    def _z_act(i, c):
        act_row[pl.ds(i * LANES, LANES)] = zero16f
        return c

    lax.fori_loop(0, NVEC, _z_act, 0)

    def _z_cidx(i, c):
        cidx_v[pl.ds(i * LANES, LANES)] = zero16i
        return c

    lax.fori_loop(0, CAP // LANES, _z_cidx, 0)
    selidx_v[pl.ds(0, LANES)] = zero16i
    selidx_v[pl.ds(LANES, LANES)] = zero16i
    selidx_v[pl.ds(2 * LANES, LANES)] = zero16i
    selidx_v[pl.ds(3 * LANES, LANES)] = zero16i
    selidx_p[pl.ds(0, LANES)] = zero16i
    selidx_p[pl.ds(LANES, LANES)] = zero16i
    selidx_p[pl.ds(2 * LANES, LANES)] = zero16i
    selidx_p[pl.ds(3 * LANES, LANES)] = zero16i

    def process(row_v, r, first):
        def p1(b, c):
            acc = row_v[pl.ds(b * 128, LANES)]
            for k in range(1, 8):
                acc = jnp.maximum(acc, row_v[pl.ds(b * 128 + k * LANES, LANES)])
            bmax_v[pl.ds(b * LANES, LANES)] = acc
            return c

        lax.fori_loop(0, NVEC // 8, p1, 0)

        def p2(g, c):
            acc = bmax_v[pl.ds(g * 256, LANES)]
            for k in range(1, 16):
                acc = jnp.maximum(acc, bmax_v[pl.ds(g * 256 + k * LANES, LANES)])
            gk_v[pl.ds(g * LANES, LANES)] = _mkey(acc)
            return c

        lax.fori_loop(0, GVREGS, p2, 0)

        def bs1(_, lohi):
            lo, hi = lohi
            mid = lo + ((hi - lo) >> 1) + np.uint32(1)
            cvec = zero16i
            for j in range(GVREGS):
                m = gk_v[pl.ds(j * LANES, LANES)] >= mid
                cvec = cvec + jnp.where(m, 1, 0)
            c = jnp.sum(cvec)
            ok = c >= TOPK
            return (jnp.where(ok, mid, lo), jnp.where(ok, hi, mid - 1))

        tau_m, _ = lax.fori_loop(
            0, 32, bs1, (np.uint32(0), np.uint32(0xFFFFFFFF))
        )
        tau_f = _inv_key(tau_m)

        def ext(b, pos):
            accb = bmax_v[pl.ds(b * LANES, LANES)]
            pc = plsc.all_reduce_population_count(accb >= tau_f)[0]

            def taken(p):
                for k in range(8):
                    off = b * 128 + k * LANES
                    v = row_v[pl.ds(off, LANES)]
                    mm = v >= tau_f
                    iv = lanes + off
                    plsc.store_compressed(cidx_v.at[pl.ds(p, LANES)], iv, mask=mm)
                    c = plsc.all_reduce_population_count(mm)[0]
                    p = jnp.minimum(p + c, CAP - LANES)
                return p

            return lax.cond(pc > 0, taken, lambda p: p, pos)

        pos = lax.fori_loop(0, NVEC // 8, ext, 0)
        vn = (pos + LANES - 1) // LANES

        def mk(j, c):
            iv = cidx_v[pl.ds(j * LANES, LANES)]
            vals = plsc.load_gather(row_v, [iv])
            valid = (lanes + j * LANES) < pos
            ckey_v[pl.ds(j * LANES, LANES)] = jnp.where(
                valid, _mkey(vals), np.uint32(0)
            )
            return c

        lax.fori_loop(0, vn, mk, 0)

        def bs2(_, lohi):
            lo, hi = lohi

            def cnt(j, acc):
                m = ckey_v[pl.ds(j * LANES, LANES)] >= lo + ((hi - lo) >> 1) + np.uint32(1)
                return acc + jnp.where(m, 1, 0)

            mid = lo + ((hi - lo) >> 1) + np.uint32(1)
            c = jnp.sum(lax.fori_loop(0, vn, cnt, zero16i))
            ok = c >= TOPK
            return (jnp.where(ok, mid, lo), jnp.where(ok, hi, mid - 1))

        tau_s, _ = lax.fori_loop(
            0, 32, bs2, (np.uint32(0), np.uint32(0xFFFFFFFF))
        )

        def cgt(j, acc):
            m = ckey_v[pl.ds(j * LANES, LANES)] > tau_s
            return acc + jnp.where(m, 1, 0)

        gt_cnt = jnp.sum(lax.fori_loop(0, vn, cgt, zero16i))
        need = TOPK - gt_cnt

        def selp(j, carry2):
            outpos, eqc = carry2
            kk = ckey_v[pl.ds(j * LANES, LANES)]
            gt = kk > tau_s
            eq = kk == tau_s
            ecs = plsc.cumsum(jnp.where(eq, 1, 0))
            take = eq & ((ecs + eqc) <= need)
            sel = gt | take
            iv = cidx_v[pl.ds(j * LANES, LANES)]
            plsc.store_compressed(selidx_v.at[pl.ds(outpos, LANES)], iv, mask=sel)
            outpos = outpos + plsc.all_reduce_population_count(sel)[0]
            eqc = eqc + ecs[15]
            return (outpos, eqc)

        lax.fori_loop(0, vn, selp, (0, 0))

        def _wait_out():
            pltpu.make_async_copy(act_row, acts.at[r], sem_o).wait()

        lax.cond(first, lambda: None, _wait_out)
        for j in range(TOPK // LANES):
            ivp = selidx_p[pl.ds(j * LANES, LANES)]
            plsc.store_scatter(act_row, [ivp], zero16f)

        for j in range(TOPK // LANES):
            iv = selidx_v[pl.ds(j * LANES, LANES)]
            vv = plsc.load_gather(row_v, [iv])
            plsc.store_scatter(act_row, [iv], vv)
        pltpu.make_async_copy(act_row, acts.at[r], sem_o).start()
        for j in range(TOPK // LANES):
            selidx_p[pl.ds(j * LANES, LANES)] = selidx_v[pl.ds(j * LANES, LANES)]

    def _start(r, row_v, sem):
        pltpu.make_async_copy(pre.at[r], row_v, sem).start()

    def _wait(r, row_v, sem):
        pltpu.make_async_copy(pre.at[r], row_v, sem).wait()

    _start(r0, row_a, sem_a)
    _start(r0 + 1, row_b, sem_b)

    def pair(i, c):
        ra = r0 + 2 * i
        _wait(ra, row_a, sem_a)
        process(row_a, ra, i == 0)
        lax.cond(
            2 * i + 2 < ROWS_PER_W,
            lambda: _start(ra + 2, row_a, sem_a),
            lambda: None,
        )
        rb = ra + 1
        _wait(rb, row_b, sem_b)
        process(row_b, rb, False)
        lax.cond(
            2 * i + 3 < ROWS_PER_W,
            lambda: _start(rb + 2, row_b, sem_b),
            lambda: None,
        )
        return c

    lax.fori_loop(0, ROWS_PER_W // 2, pair, 0)
    pltpu.make_async_copy(act_row, acts.at[r0 + ROWS_PER_W - 1], sem_o).wait()


def _sc_select(pre):
    mesh = plsc.VectorSubcoreMesh(
        core_axis_name="c", subcore_axis_name="s"
    )
    return pl.kernel(
        _sc_body,
        out_type=jax.ShapeDtypeStruct((TOKENS, FACTORS), jnp.float32),
        mesh=mesh,
        compiler_params=pltpu.CompilerParams(needs_layout_passes=False),
        scratch_types=[
            pltpu.VMEM((FACTORS,), jnp.float32),
            pltpu.VMEM((FACTORS,), jnp.float32),
            pltpu.VMEM((NVEC // 8 * LANES,), jnp.float32),
            pltpu.VMEM((GROUPS,), jnp.uint32),
            pltpu.VMEM((CAP,), jnp.int32),
            pltpu.VMEM((CAP,), jnp.uint32),
            pltpu.VMEM((TOPK,), jnp.int32),
            pltpu.VMEM((TOPK,), jnp.int32),
            pltpu.VMEM((FACTORS,), jnp.float32),
            pltpu.SemaphoreType.DMA,
            pltpu.SemaphoreType.DMA,
            pltpu.SemaphoreType.DMA,
        ],
    )(pre)


def kernel(x, W_enc, W_dec, b_pre):
    pre = _pre_acts(x, W_enc, b_pre)
    acts = _sc_select(pre)
    x_hat = _decode(acts, W_dec, b_pre)
    return (x_hat, acts)
